# baseline (speedup 1.0000x reference)
import functools

import jax
import jax.numpy as jnp
import numpy as np
from jax.experimental import pallas as pl
from jax.experimental.pallas import tpu as pltpu

_EPS = 1e-5
_B = 16
_N = 1024
_NEG_FILL = -1e10
_MASK_VAL = -1e9


def _fold_mlp(layers):
    Ws, bs = [], []
    s_prev = t_prev = None
    for layer in layers:
        W, b = layer["lin"]["W"], layer["lin"]["b"]
        if s_prev is not None:
            b = b + t_prev @ W
            W = s_prev[:, None] * W
        Ws.append(W)
        bs.append(b)
        bn = layer["bn"]
        s = bn["gamma"] / jnp.sqrt(bn["var"] + _EPS)
        t = bn["beta"] - bn["mean"] * s
        s_prev, t_prev = s, t
    return Ws, bs, s_prev, t_prev


def _dot(a, b):
    return jax.lax.dot_general(a, b, (((1,), (0,)), ((), ())),
                               preferred_element_type=jnp.float32)


def _fps_body(m, pos_ref, out_ref, dmin_ref):
    nc, _, n = pos_ref.shape
    pos = pos_ref[...]
    iota_n = jax.lax.broadcasted_iota(jnp.int32, (nc, 1, n), 2)
    iota_m = jax.lax.broadcasted_iota(jnp.int32, (nc, 1, m), 2)
    last0 = pos[:, :, 0:1]
    out_ref[...] = last0 * (iota_m == 0).astype(jnp.float32)
    dmin_ref[...] = jnp.full((nc, 1, n), jnp.inf, jnp.float32)

    def body(i, last):
        p = pos_ref[...]
        diff = p - last
        d = jnp.sum(diff * diff, axis=1, keepdims=True)
        dmin = jnp.minimum(dmin_ref[...], d)
        dmin_ref[...] = dmin
        rmax = jnp.max(dmin, axis=2, keepdims=True)
        idx = jnp.min(jnp.where(dmin == rmax, iota_n, n), axis=2,
                      keepdims=True)
        sel = (iota_n == idx).astype(jnp.float32)
        newlast = jnp.sum(p * sel, axis=2, keepdims=True)
        out_ref[...] += newlast * (iota_m == i).astype(jnp.float32)
        return newlast

    jax.lax.fori_loop(1, m, body, last0)


def _fps(posT, m):
    nc, _, n = posT.shape
    return pl.pallas_call(
        functools.partial(_fps_body, m),
        out_shape=jax.ShapeDtypeStruct((nc, 3, m), jnp.float32),
        scratch_shapes=[pltpu.VMEM((nc, 1, n), jnp.float32)],
    )(posT)


def _sa_body(r2, k_nbr, mc, posT_ref, ctr_ref, thi_ref, tlo_ref, ctradj_ref,
             w1_ref, b1_ref, w2_ref, b2_ref, w3_ref, b3_ref, s_ref, t_ref,
             out_ref, neg_ref, cnt_ref, idx_ref):
    m = ctr_ref.shape[1]
    n = posT_ref.shape[2]
    cpad = thi_ref.shape[2]
    posT = posT_ref[0]
    ctr = ctr_ref[0]
    d2 = jnp.zeros((m, n), jnp.float32)
    for c in range(3):
        diff = ctr[:, c:c + 1] - posT[c:c + 1, :]
        d2 = d2 + diff * diff
    within = d2 <= r2
    neg_ref[...] = jnp.where(within, -d2, jnp.float32(_NEG_FILL))
    cnt_ref[...] = jnp.sum(within.astype(jnp.int32), axis=1, keepdims=True)
    iota_n = jax.lax.broadcasted_iota(jnp.int32, (m, n), 1)

    def sel_round(k, _):
        neg = neg_ref[...]
        rmax = jnp.max(neg, axis=1, keepdims=True)
        eq = jnp.logical_and(neg == rmax, rmax > jnp.float32(_MASK_VAL))
        idx = jnp.min(jnp.where(eq, iota_n, n), axis=1, keepdims=True)
        neg_ref[...] = jnp.where(iota_n == idx, jnp.float32(_NEG_FILL), neg)
        idx_ref[pl.ds(k, 1)] = idx[None]
        return 0

    jax.lax.fori_loop(0, k_nbr, sel_round, 0)

    w1, b1 = w1_ref[...], b1_ref[...]
    w2, b2 = w2_ref[...], b2_ref[...]
    w3, b3 = w3_ref[...], b3_ref[...]
    s, t = s_ref[...], t_ref[...]
    kio = jax.lax.broadcasted_iota(jnp.int32, (k_nbr, 1, 1), 0)

    iota_p = jax.lax.broadcasted_iota(jnp.int32, (k_nbr * mc, n), 1)

    def chunk(j, _):
        jm = pl.multiple_of(j * mc, mc)
        idxc = idx_ref[:, pl.ds(jm, mc), :].reshape(k_nbr * mc, 1)
        oh = (iota_p == idxc).astype(jnp.bfloat16)
        g = _dot(oh, thi_ref[0]) + _dot(oh, tlo_ref[0])
        adj = ctradj_ref[0, pl.ds(jm, mc), :].reshape(1, mc, cpad)
        z = (g.reshape(k_nbr, mc, cpad) - adj).reshape(k_nbr * mc, cpad)
        h = jnp.maximum(_dot(z, w1) + b1, 0.0)
        h = jnp.maximum(_dot(h, w2) + b2, 0.0)
        h = jnp.maximum(_dot(h, w3) + b3, 0.0)
        h = h * s + t
        h3 = h.reshape(k_nbr, mc, h.shape[1])
        cntc = cnt_ref[pl.ds(jm, mc), :].reshape(1, mc, 1)
        h3 = jnp.where(kio < cntc, h3, jnp.float32(_MASK_VAL))
        out_ref[0, pl.ds(jm, mc), :] = jnp.max(h3, axis=0)
        return 0

    jax.lax.fori_loop(0, m // mc, chunk, 0)


def _sa(posT, ctr, table, ctradj, Ws, bs, s, t, r2, k_nbr=64, mc=32):
    nc, _, n = posT.shape
    m = ctr.shape[1]
    cpad = table.shape[2]
    f_out = Ws[2].shape[1]
    thi = table.astype(jnp.bfloat16)
    tlo = (table - thi.astype(jnp.float32)).astype(jnp.bfloat16)
    bcast = pl.BlockSpec(None, lambda c: (0, 0))
    return pl.pallas_call(
        functools.partial(_sa_body, np.float32(r2), k_nbr, mc),
        grid=(nc,),
        in_specs=[
            pl.BlockSpec((1, 3, n), lambda c: (c, 0, 0)),
            pl.BlockSpec((1, m, 3), lambda c: (c, 0, 0)),
            pl.BlockSpec((1, n, cpad), lambda c: (c, 0, 0)),
            pl.BlockSpec((1, n, cpad), lambda c: (c, 0, 0)),
            pl.BlockSpec((1, m, cpad), lambda c: (c, 0, 0)),
            bcast, bcast, bcast, bcast, bcast, bcast, bcast, bcast,
        ],
        out_specs=pl.BlockSpec((1, m, f_out), lambda c: (c, 0, 0)),
        out_shape=jax.ShapeDtypeStruct((nc, m, f_out), jnp.float32),
        scratch_shapes=[
            pltpu.VMEM((m, n), jnp.float32),
            pltpu.VMEM((m, 1), jnp.int32),
            pltpu.VMEM((k_nbr, m, 1), jnp.int32),
        ],
    )(posT, ctr, thi, tlo, ctradj, Ws[0], bs[0][None, :], Ws[1],
      bs[1][None, :], Ws[2], bs[2][None, :], s[None, :], t[None, :])


def _gsa_body(z_ref, w1_ref, b1_ref, w2_ref, b2_ref, w3_ref, b3_ref,
              s_ref, t_ref, out_ref):
    z = z_ref[0]
    h = jnp.maximum(_dot(z, w1_ref[...]) + b1_ref[...], 0.0)
    h = jnp.maximum(_dot(h, w2_ref[...]) + b2_ref[...], 0.0)
    h = jnp.maximum(_dot(h, w3_ref[...]) + b3_ref[...], 0.0)
    h = h * s_ref[...] + t_ref[...]
    out_ref[0] = jnp.max(h, axis=0, keepdims=True)


def _gsa(z, Ws, bs, s, t):
    nc, npts, cpad = z.shape
    f_out = Ws[2].shape[1]
    bcast = pl.BlockSpec(None, lambda c: (0, 0))
    return pl.pallas_call(
        _gsa_body,
        grid=(nc,),
        in_specs=[pl.BlockSpec((1, npts, cpad), lambda c: (c, 0, 0)),
                  bcast, bcast, bcast, bcast, bcast, bcast, bcast, bcast],
        out_specs=pl.BlockSpec((1, 1, f_out), lambda c: (c, 0, 0)),
        out_shape=jax.ShapeDtypeStruct((nc, 1, f_out), jnp.float32),
    )(z, Ws[0], bs[0][None, :], Ws[1], bs[1][None, :], Ws[2], bs[2][None, :],
      s[None, :], t[None, :])


def _head_body(h_ref, w1_ref, b1_ref, w2_ref, b2_ref, w3_ref, b3_ref,
               out_ref):
    h = h_ref[...]
    h = h[:_B] + h[_B:]
    h = jnp.maximum(_dot(h, w1_ref[...]) + b1_ref[...], 0.0)
    h = jnp.maximum(_dot(h, w2_ref[...]) + b2_ref[...], 0.0)
    logits = _dot(h, w3_ref[...]) + b3_ref[...]
    mx = jnp.max(logits, axis=1, keepdims=True)
    sh = logits - mx
    lse = jnp.log(jnp.sum(jnp.exp(sh), axis=1, keepdims=True))
    out_ref[...] = sh - lse


def _head(h, p1, p2, p3):
    return pl.pallas_call(
        _head_body,
        out_shape=jax.ShapeDtypeStruct((_B, 128), jnp.float32),
    )(h, p1["W"], p1["b"][None, :], p2["W"], p2["b"][None, :],
      p3["W"], p3["b"][None, :])


def kernel(x, x2, batch, params):
    pos = jnp.concatenate([x.reshape(_B, _N, 3), x2.reshape(_B, _N, 3)], 0)
    nc = 2 * _B
    posT = jnp.transpose(pos, (0, 2, 1))

    W1s, b1s, s1, t1 = _fold_mlp(params["mlp1"])
    W2s, b2s, s2, t2 = _fold_mlp(params["mlp2"])
    W3s, b3s, s3, t3 = _fold_mlp(params["mlp3"])

    c1T = _fps(posT, _N // 2)
    ctr1 = jnp.transpose(c1T, (0, 2, 1))
    table1 = jnp.pad(pos, ((0, 0), (0, 0), (0, 5)))
    ctradj1 = jnp.pad(ctr1, ((0, 0), (0, 0), (0, 5)))
    W1p = [jnp.pad(W1s[0], ((0, 5), (0, 0))), W1s[1], W1s[2]]
    h1 = _sa(posT, ctr1, table1, ctradj1, W1p, b1s, s1, t1, 0.2 * 0.2)

    c2T = _fps(c1T, _N // 8)
    ctr2 = jnp.transpose(c2T, (0, 2, 1))
    table2 = jnp.pad(jnp.concatenate([h1, ctr1], axis=2),
                     ((0, 0), (0, 0), (0, 5)))
    ctradj2 = jnp.pad(jnp.pad(ctr2, ((0, 0), (0, 0), (128, 0))),
                      ((0, 0), (0, 0), (0, 5)))
    W2p = [jnp.pad(W2s[0], ((0, 5), (0, 0))), W2s[1], W2s[2]]
    h2 = _sa(c1T, ctr2, table2, ctradj2, W2p, b2s, s2, t2, 0.4 * 0.4)

    z3 = jnp.pad(jnp.concatenate([h2, ctr2], axis=2),
                 ((0, 0), (0, 0), (0, 384 - 259)))
    W3p = [jnp.pad(W3s[0], ((0, 384 - 259), (0, 0))), W3s[1], W3s[2]]
    h3 = _gsa(z3, W3p, b3s, s3, t3)[:, 0, :]

    lin3 = params["lin3"]
    l3 = {"W": jnp.pad(lin3["W"], ((0, 0), (0, 118))),
          "b": jnp.pad(lin3["b"], (0, 118),
                       constant_values=np.float32(-1e30))}
    out = _head(h3, params["lin1"], params["lin2"], l3)
    return out[:, :10]


# speedup vs baseline: 1.4478x; 1.4478x over previous
import functools

import jax
import jax.numpy as jnp
import numpy as np
from jax.experimental import pallas as pl
from jax.experimental.pallas import tpu as pltpu

_EPS = 1e-5
_B = 16
_N = 1024
_NEG_FILL = -1e10
_MASK_VAL = -1e9


def _fold_mlp(layers):
    Ws, bs = [], []
    s_prev = t_prev = None
    for layer in layers:
        W, b = layer["lin"]["W"], layer["lin"]["b"]
        if s_prev is not None:
            b = b + t_prev @ W
            W = s_prev[:, None] * W
        Ws.append(W)
        bs.append(b)
        bn = layer["bn"]
        s = bn["gamma"] / jnp.sqrt(bn["var"] + _EPS)
        t = bn["beta"] - bn["mean"] * s
        s_prev, t_prev = s, t
    return Ws, bs, s_prev, t_prev


def _dot(a, b):
    return jax.lax.dot_general(a, b, (((1,), (0,)), ((), ())),
                               preferred_element_type=jnp.float32)


def _fps_body(m, pos_ref, out_ref, dmin_ref):
    nc, _, n = pos_ref.shape
    pos = pos_ref[...]
    iota_n = jax.lax.broadcasted_iota(jnp.int32, (nc, 1, n), 2)
    iota_m = jax.lax.broadcasted_iota(jnp.int32, (nc, 1, m), 2)
    last0 = pos[:, :, 0:1]
    out_ref[...] = last0 * (iota_m == 0).astype(jnp.float32)
    dmin_ref[...] = jnp.full((nc, 1, n), jnp.inf, jnp.float32)

    def body(i, last):
        p = pos_ref[...]
        diff = p - last
        d = jnp.sum(diff * diff, axis=1, keepdims=True)
        dmin = jnp.minimum(dmin_ref[...], d)
        dmin_ref[...] = dmin
        rmax = jnp.max(dmin, axis=2, keepdims=True)
        idx = jnp.min(jnp.where(dmin == rmax, iota_n, n), axis=2,
                      keepdims=True)
        sel = (iota_n == idx).astype(jnp.float32)
        newlast = jnp.sum(p * sel, axis=2, keepdims=True)
        out_ref[...] += newlast * (iota_m == i).astype(jnp.float32)
        return newlast

    jax.lax.fori_loop(1, m, body, last0)


def _fps(posT, m):
    nc, _, n = posT.shape
    return pl.pallas_call(
        functools.partial(_fps_body, m),
        out_shape=jax.ShapeDtypeStruct((nc, 3, m), jnp.float32),
        scratch_shapes=[pltpu.VMEM((nc, 1, n), jnp.float32)],
    )(posT)


def _sa_body(kw, k_nbr, mc, posT_ref, ctr_ref, thi_ref, tlo_ref, ctradj_ref,
             lt_ref, w1_ref, b1_ref, w2_ref, b2_ref, w3_ref, b3_ref, s_ref,
             t_ref, out_ref, key_ref, cnt_ref):
    m = ctr_ref.shape[1]
    n = posT_ref.shape[2]
    cpad = thi_ref.shape[2]
    posT = posT_ref[0]
    ctr = ctr_ref[0]
    d2 = jnp.zeros((m, n), jnp.float32)
    for c in range(3):
        diff = ctr[:, c:c + 1] - posT[c:c + 1, :]
        d2 = d2 + diff * diff
    bits = jax.lax.bitcast_convert_type(-d2, jnp.int32)
    key = jnp.bitwise_xor(bits, jnp.int32(-1))
    wmask = key >= jnp.int32(kw)
    cnt_ref[...] = jnp.sum(wmask.astype(jnp.int32), axis=1, keepdims=True)
    t = jnp.zeros((m, 1), jnp.int32)
    for b in range(30, -1, -1):
        cand = jnp.bitwise_or(t, jnp.int32(1 << b))
        c = jnp.sum((key >= cand).astype(jnp.int32), axis=1, keepdims=True)
        t = jnp.where(c >= k_nbr, cand, t)
    sel = jnp.logical_and(key >= t, wmask)
    rank = _dot(sel.astype(jnp.bfloat16), lt_ref[...]).astype(jnp.int32)
    key_ref[...] = jnp.where(sel, rank, -1)

    w1, b1 = w1_ref[...], b1_ref[...]
    w2, b2 = w2_ref[...], b2_ref[...]
    w3, b3 = w3_ref[...], b3_ref[...]
    s, t = s_ref[...], t_ref[...]
    kio = jax.lax.broadcasted_iota(jnp.int32, (k_nbr, 1, 1), 0)

    def chunk(j, _):
        jm = pl.multiple_of(j * mc, mc)
        sc = key_ref[pl.ds(jm, mc), :].reshape(1, mc, n)
        oh = (kio == sc).astype(jnp.bfloat16).reshape(k_nbr * mc, n)
        g = _dot(oh, thi_ref[0]) + _dot(oh, tlo_ref[0])
        adj = ctradj_ref[0, pl.ds(jm, mc), :].reshape(1, mc, cpad)
        z = (g.reshape(k_nbr, mc, cpad) - adj).reshape(k_nbr * mc, cpad)
        h = jnp.maximum(_dot(z, w1) + b1, 0.0)
        h = jnp.maximum(_dot(h, w2) + b2, 0.0)
        h = jnp.maximum(_dot(h, w3) + b3, 0.0)
        h = h * s + t
        h3 = h.reshape(k_nbr, mc, h.shape[1])
        cntc = cnt_ref[pl.ds(jm, mc), :].reshape(1, mc, 1)
        h3 = jnp.where(kio < cntc, h3, jnp.float32(_MASK_VAL))
        out_ref[0, pl.ds(jm, mc), :] = jnp.max(h3, axis=0)
        return 0

    jax.lax.fori_loop(0, m // mc, chunk, 0)


def _sa(posT, ctr, table, ctradj, Ws, bs, s, t, r2, k_nbr=64, mc=32):
    nc, _, n = posT.shape
    m = ctr.shape[1]
    cpad = table.shape[2]
    f_out = Ws[2].shape[1]
    thi = table.astype(jnp.bfloat16)
    tlo = (table - thi.astype(jnp.float32)).astype(jnp.bfloat16)
    lt = jnp.triu(jnp.ones((n, n), jnp.bfloat16), k=1)
    kw = int(np.invert(np.float32(-np.float32(r2)).view(np.int32)))
    bcast = pl.BlockSpec(None, lambda c: (0, 0))
    return pl.pallas_call(
        functools.partial(_sa_body, kw, k_nbr, mc),
        grid=(nc,),
        in_specs=[
            pl.BlockSpec((1, 3, n), lambda c: (c, 0, 0)),
            pl.BlockSpec((1, m, 3), lambda c: (c, 0, 0)),
            pl.BlockSpec((1, n, cpad), lambda c: (c, 0, 0)),
            pl.BlockSpec((1, n, cpad), lambda c: (c, 0, 0)),
            pl.BlockSpec((1, m, cpad), lambda c: (c, 0, 0)),
            bcast, bcast, bcast, bcast, bcast, bcast, bcast, bcast, bcast,
        ],
        out_specs=pl.BlockSpec((1, m, f_out), lambda c: (c, 0, 0)),
        out_shape=jax.ShapeDtypeStruct((nc, m, f_out), jnp.float32),
        scratch_shapes=[
            pltpu.VMEM((m, n), jnp.int32),
            pltpu.VMEM((m, 1), jnp.int32),
        ],
    )(posT, ctr, thi, tlo, ctradj, lt, Ws[0], bs[0][None, :], Ws[1],
      bs[1][None, :], Ws[2], bs[2][None, :], s[None, :], t[None, :])


def _gsa_body(z_ref, w1_ref, b1_ref, w2_ref, b2_ref, w3_ref, b3_ref,
              s_ref, t_ref, out_ref):
    z = z_ref[0]
    h = jnp.maximum(_dot(z, w1_ref[...]) + b1_ref[...], 0.0)
    h = jnp.maximum(_dot(h, w2_ref[...]) + b2_ref[...], 0.0)
    h = jnp.maximum(_dot(h, w3_ref[...]) + b3_ref[...], 0.0)
    h = h * s_ref[...] + t_ref[...]
    out_ref[0] = jnp.max(h, axis=0, keepdims=True)


def _gsa(z, Ws, bs, s, t):
    nc, npts, cpad = z.shape
    f_out = Ws[2].shape[1]
    bcast = pl.BlockSpec(None, lambda c: (0, 0))
    return pl.pallas_call(
        _gsa_body,
        grid=(nc,),
        in_specs=[pl.BlockSpec((1, npts, cpad), lambda c: (c, 0, 0)),
                  bcast, bcast, bcast, bcast, bcast, bcast, bcast, bcast],
        out_specs=pl.BlockSpec((1, 1, f_out), lambda c: (c, 0, 0)),
        out_shape=jax.ShapeDtypeStruct((nc, 1, f_out), jnp.float32),
    )(z, Ws[0], bs[0][None, :], Ws[1], bs[1][None, :], Ws[2], bs[2][None, :],
      s[None, :], t[None, :])


def _head_body(h_ref, w1_ref, b1_ref, w2_ref, b2_ref, w3_ref, b3_ref,
               out_ref):
    h = h_ref[...]
    h = h[:_B] + h[_B:]
    h = jnp.maximum(_dot(h, w1_ref[...]) + b1_ref[...], 0.0)
    h = jnp.maximum(_dot(h, w2_ref[...]) + b2_ref[...], 0.0)
    logits = _dot(h, w3_ref[...]) + b3_ref[...]
    mx = jnp.max(logits, axis=1, keepdims=True)
    sh = logits - mx
    lse = jnp.log(jnp.sum(jnp.exp(sh), axis=1, keepdims=True))
    out_ref[...] = sh - lse


def _head(h, p1, p2, p3):
    return pl.pallas_call(
        _head_body,
        out_shape=jax.ShapeDtypeStruct((_B, 128), jnp.float32),
    )(h, p1["W"], p1["b"][None, :], p2["W"], p2["b"][None, :],
      p3["W"], p3["b"][None, :])


def kernel(x, x2, batch, params):
    pos = jnp.concatenate([x.reshape(_B, _N, 3), x2.reshape(_B, _N, 3)], 0)
    nc = 2 * _B
    posT = jnp.transpose(pos, (0, 2, 1))

    W1s, b1s, s1, t1 = _fold_mlp(params["mlp1"])
    W2s, b2s, s2, t2 = _fold_mlp(params["mlp2"])
    W3s, b3s, s3, t3 = _fold_mlp(params["mlp3"])

    c1T = _fps(posT, _N // 2)
    ctr1 = jnp.transpose(c1T, (0, 2, 1))
    table1 = jnp.pad(pos, ((0, 0), (0, 0), (0, 5)))
    ctradj1 = jnp.pad(ctr1, ((0, 0), (0, 0), (0, 5)))
    W1p = [jnp.pad(W1s[0], ((0, 5), (0, 0))), W1s[1], W1s[2]]
    h1 = _sa(posT, ctr1, table1, ctradj1, W1p, b1s, s1, t1, 0.2 * 0.2)

    c2T = _fps(c1T, _N // 8)
    ctr2 = jnp.transpose(c2T, (0, 2, 1))
    table2 = jnp.pad(jnp.concatenate([h1, ctr1], axis=2),
                     ((0, 0), (0, 0), (0, 5)))
    ctradj2 = jnp.pad(jnp.pad(ctr2, ((0, 0), (0, 0), (128, 0))),
                      ((0, 0), (0, 0), (0, 5)))
    W2p = [jnp.pad(W2s[0], ((0, 5), (0, 0))), W2s[1], W2s[2]]
    h2 = _sa(c1T, ctr2, table2, ctradj2, W2p, b2s, s2, t2, 0.4 * 0.4)

    z3 = jnp.pad(jnp.concatenate([h2, ctr2], axis=2),
                 ((0, 0), (0, 0), (0, 384 - 259)))
    W3p = [jnp.pad(W3s[0], ((0, 384 - 259), (0, 0))), W3s[1], W3s[2]]
    h3 = _gsa(z3, W3p, b3s, s3, t3)[:, 0, :]

    lin3 = params["lin3"]
    l3 = {"W": jnp.pad(lin3["W"], ((0, 0), (0, 118))),
          "b": jnp.pad(lin3["b"], (0, 118),
                       constant_values=np.float32(-1e30))}
    out = _head(h3, params["lin1"], params["lin2"], l3)
    return out[:, :10]
